# Initial kernel scaffold; baseline (speedup 1.0000x reference)
import jax
import jax.numpy as jnp
from jax.experimental import pallas as pl


def kernel(points):
    raise NotImplementedError("write your pallas kernel here")


# speedup vs baseline: 3.9205x; 3.9205x over previous
import functools

import jax
import jax.numpy as jnp
from jax import lax
from jax.experimental import pallas as pl
from jax.experimental.pallas import tpu as pltpu
from jax.experimental.pallas import tpu_sc as plsc

_G = 512
_K = 32


def _fps_body(x_ref, y_ref, z_ref, cx_ref, cy_ref, cz_ref, md_ref):
    B, N = x_ref.shape
    G = cx_ref.shape[1]
    x = x_ref[...]
    y = y_ref[...]
    z = z_ref[...]
    lanes = lax.broadcasted_iota(jnp.int32, (B, N), 1)
    glanes = lax.broadcasted_iota(jnp.int32, (B, G), 1)
    md_ref[...] = jnp.full((B, N), 1e10, jnp.float32)

    def body(i, far):
        m = lanes == far
        cx = jnp.sum(jnp.where(m, x, 0.0), axis=1, keepdims=True)
        cy = jnp.sum(jnp.where(m, y, 0.0), axis=1, keepdims=True)
        cz = jnp.sum(jnp.where(m, z, 0.0), axis=1, keepdims=True)
        gm = glanes == i
        cx_ref[...] = jnp.where(gm, cx, cx_ref[...])
        cy_ref[...] = jnp.where(gm, cy, cy_ref[...])
        cz_ref[...] = jnp.where(gm, cz, cz_ref[...])
        dx = x - cx
        dy = y - cy
        dz = z - cz
        d = dx * dx + dy * dy + dz * dz
        md = jnp.minimum(md_ref[...], d)
        md_ref[...] = md
        mx = jnp.max(md, axis=1, keepdims=True)
        far = jnp.min(jnp.where(md == mx, lanes, N), axis=1, keepdims=True)
        return far.astype(jnp.int32)

    lax.fori_loop(0, G, body, jnp.zeros((B, 1), jnp.int32))


def _fps(x, y, z, g):
    B, N = x.shape
    out = jax.ShapeDtypeStruct((B, g), jnp.float32)
    return pl.pallas_call(
        _fps_body,
        out_shape=[out, out, out],
        scratch_shapes=[pltpu.VMEM((B, N), jnp.float32)],
    )(x, y, z)


def _knn_body(x_ref, y_ref, z_ref, cx_ref, cy_ref, cz_ref, idx_ref, d2_ref):
    N = x_ref.shape[-1]
    R, K = idx_ref.shape[2], idx_ref.shape[3]
    x = x_ref[0]
    y = y_ref[0]
    z = z_ref[0]
    cx = cx_ref[0, 0]
    cy = cy_ref[0, 0]
    cz = cz_ref[0, 0]
    dx = cx - x
    dy = cy - y
    dz = cz - z
    d2_ref[...] = dx * dx + dy * dy + dz * dz
    lanes = lax.broadcasted_iota(jnp.int32, (R, N), 1)
    klanes = lax.broadcasted_iota(jnp.int32, (R, K), 1)

    def body(k, _):
        d2 = d2_ref[...]
        m = jnp.min(d2, axis=1, keepdims=True)
        j = jnp.min(jnp.where(d2 == m, lanes, N), axis=1, keepdims=True)
        j = j.astype(jnp.int32)
        idx_ref[0, 0] = jnp.where(klanes == k, j, idx_ref[0, 0])
        d2_ref[...] = jnp.where(lanes == j, jnp.float32(jnp.inf), d2)
        return 0

    lax.fori_loop(0, K, body, 0)


def _knn(x, y, z, cx, cy, cz, k, rows=8):
    B, N = x.shape
    g = cx.shape[1]
    nb = g // rows
    x3 = x.reshape(B, 1, N)
    y3 = y.reshape(B, 1, N)
    z3 = z.reshape(B, 1, N)
    cx4 = cx.reshape(B, nb, rows, 1)
    cy4 = cy.reshape(B, nb, rows, 1)
    cz4 = cz.reshape(B, nb, rows, 1)
    pt_spec = pl.BlockSpec((1, 1, N), lambda b, i: (b, 0, 0))
    c_spec = pl.BlockSpec((1, 1, rows, 1), lambda b, i: (b, i, 0, 0))
    idx = pl.pallas_call(
        _knn_body,
        grid=(B, nb),
        in_specs=[pt_spec, pt_spec, pt_spec, c_spec, c_spec, c_spec],
        out_specs=pl.BlockSpec((1, 1, rows, k), lambda b, i: (b, i, 0, 0)),
        out_shape=jax.ShapeDtypeStruct((B, nb, rows, k), jnp.int32),
        scratch_shapes=[pltpu.VMEM((rows, N), jnp.float32)],
    )(x3, y3, z3, cx4, cy4, cz4)
    return idx.reshape(B, g, k)


def _gather_groups(points_flat, idxflat, c4, n_chunks, chunk16):
    B = points_flat.shape[0]
    nf = points_flat.shape[1]
    nw = 32
    per_w = n_chunks * 16
    groups_per_w = c4.shape[0] // nw
    mesh = plsc.VectorSubcoreMesh(core_axis_name="c", subcore_axis_name="s")

    @functools.partial(
        pl.kernel,
        out_type=jax.ShapeDtypeStruct((nw * per_w,), jnp.float32),
        mesh=mesh,
        compiler_params=pltpu.CompilerParams(needs_layout_passes=False),
        scratch_types=[
            pltpu.VMEM((per_w,), jnp.int32),
            pltpu.VMEM((nf,), jnp.float32),
            pltpu.VMEM((groups_per_w, 16), jnp.float32),
            pltpu.VMEM((per_w,), jnp.float32),
        ],
    )
    def sc_gather(pts_hbm, idxflat_hbm, c4_hbm, out_hbm, idx_v, pts_v, c4_v, out_v):
        wid = lax.axis_index("s") * 2 + lax.axis_index("c")
        b = wid // (nw // B)
        base = wid * per_w
        pltpu.sync_copy(idxflat_hbm.at[pl.ds(base, per_w)], idx_v)
        pltpu.sync_copy(pts_hbm.at[b], pts_v)
        pltpu.sync_copy(c4_hbm.at[pl.ds(wid * groups_per_w, groups_per_w)], c4_v)

        def body(c, _):
            ids = idx_v[pl.ds(c * 16, 16)]
            val = plsc.load_gather(pts_v, [ids])
            out_v[pl.ds(c * 16, 16)] = val - c4_v[c // chunk16]
            return 0

        lax.fori_loop(0, n_chunks, body, 0)
        pltpu.sync_copy(out_v, out_hbm.at[pl.ds(base, per_w)])

    return sc_gather(points_flat, idxflat, c4)


def kernel(points):
    B, N, C = points.shape
    x = points[:, :, 0]
    y = points[:, :, 1]
    z = points[:, :, 2]

    cx, cy, cz = _fps(x, y, z, _G)
    centers = jnp.stack([cx, cy, cz], axis=-1)

    idx = _knn(x, y, z, cx, cy, cz, _K)

    idxflat = (idx.reshape(B, _G * _K, 1) * 4
               + jnp.arange(4, dtype=jnp.int32)).reshape(-1)
    c4 = jnp.tile(
        jnp.concatenate([centers, jnp.zeros((B, _G, 1), jnp.float32)], axis=-1),
        (1, 1, 4),
    ).reshape(B * _G, 16)
    n_chunks = (_G // (32 // B)) * _K * 4 // 16
    flat = _gather_groups(points.reshape(B, N * 4), idxflat, c4,
                          n_chunks, _K * 4 // 16)
    groups = flat.reshape(B, _G, _K, 4)
    return groups, centers


# speedup vs baseline: 15.1313x; 3.8595x over previous
import functools

import jax
import jax.numpy as jnp
from jax import lax
from jax.experimental import pallas as pl
from jax.experimental.pallas import tpu as pltpu
from jax.experimental.pallas import tpu_sc as plsc

_G = 512
_K = 32


def _fps_body(x_ref, y_ref, z_ref, cx_ref, cy_ref, cz_ref, md_ref):
    B, N = x_ref.shape
    G = cx_ref.shape[1]
    x = x_ref[...]
    y = y_ref[...]
    z = z_ref[...]
    lanes = lax.broadcasted_iota(jnp.int32, (B, N), 1)
    glanes = lax.broadcasted_iota(jnp.int32, (B, G), 1)
    md_ref[...] = jnp.full((B, N), 1e10, jnp.float32)

    def body(i, far):
        m = lanes == far
        cx = jnp.sum(jnp.where(m, x, 0.0), axis=1, keepdims=True)
        cy = jnp.sum(jnp.where(m, y, 0.0), axis=1, keepdims=True)
        cz = jnp.sum(jnp.where(m, z, 0.0), axis=1, keepdims=True)
        gm = glanes == i
        cx_ref[...] = jnp.where(gm, cx, cx_ref[...])
        cy_ref[...] = jnp.where(gm, cy, cy_ref[...])
        cz_ref[...] = jnp.where(gm, cz, cz_ref[...])
        dx = x - cx
        dy = y - cy
        dz = z - cz
        d = dx * dx + dy * dy + dz * dz
        md = jnp.minimum(md_ref[...], d)
        md_ref[...] = md
        mx = jnp.max(md, axis=1, keepdims=True)
        far = jnp.min(jnp.where(md == mx, lanes, N), axis=1, keepdims=True)
        return far.astype(jnp.int32)

    lax.fori_loop(0, G, body, jnp.zeros((B, 1), jnp.int32))


def _fps(x, y, z, g):
    B, N = x.shape
    out = jax.ShapeDtypeStruct((B, g), jnp.float32)
    return pl.pallas_call(
        _fps_body,
        out_shape=[out, out, out],
        scratch_shapes=[pltpu.VMEM((B, N), jnp.float32)],
    )(x, y, z)


_SEG = 256
_CPS = _SEG // 16


def _knn_gather_sc(x, y, z, w, cx, cy, cz, g, k):
    B, N = x.shape
    nw = 32
    rows_per_w = (B * g) // nw
    w_per_b = nw // B
    nseg = N // _SEG
    per_w = rows_per_w * k * 4
    mesh = plsc.VectorSubcoreMesh(core_axis_name="c", subcore_axis_name="s")
    big = jnp.int32(2 ** 30)
    inf = jnp.float32(jnp.inf)

    @functools.partial(
        pl.kernel,
        out_type=jax.ShapeDtypeStruct((nw * per_w,), jnp.float32),
        mesh=mesh,
        compiler_params=pltpu.CompilerParams(needs_layout_passes=False),
        scratch_types=[
            pltpu.VMEM((N,), jnp.float32),
            pltpu.VMEM((N,), jnp.float32),
            pltpu.VMEM((N,), jnp.float32),
            pltpu.VMEM((N,), jnp.float32),
            pltpu.VMEM((rows_per_w,), jnp.float32),
            pltpu.VMEM((rows_per_w,), jnp.float32),
            pltpu.VMEM((rows_per_w,), jnp.float32),
            pltpu.VMEM((N,), jnp.float32),
            pltpu.VMEM((nseg,), jnp.float32),
            pltpu.VMEM((k * 4,), jnp.int32),
            pltpu.VMEM((per_w,), jnp.float32),
        ],
    )
    def sc_knn(x_hbm, y_hbm, z_hbm, w_hbm, cx_hbm, cy_hbm, cz_hbm, out_hbm,
               x_v, y_v, z_v, w_v, cx_v, cy_v, cz_v, d2_v, sm_v, sel_v,
               out_v):
        wid = lax.axis_index("s") * 2 + lax.axis_index("c")
        b = wid // w_per_b
        g0 = (wid % w_per_b) * rows_per_w
        pltpu.sync_copy(x_hbm.at[b], x_v)
        pltpu.sync_copy(y_hbm.at[b], y_v)
        pltpu.sync_copy(z_hbm.at[b], z_v)
        pltpu.sync_copy(w_hbm.at[b], w_v)
        pltpu.sync_copy(cx_hbm.at[b, pl.ds(g0, rows_per_w)], cx_v)
        pltpu.sync_copy(cy_hbm.at[b, pl.ds(g0, rows_per_w)], cy_v)
        pltpu.sync_copy(cz_hbm.at[b, pl.ds(g0, rows_per_w)], cz_v)

        lane = jax.lax.iota(jnp.int32, 16)
        feat = lane % 4
        lane0 = lane == 0

        def splat_at(ref, i):
            return plsc.load_gather(ref, [jnp.full((16,), i, jnp.int32)])

        def store_at(ref, i, v):
            plsc.store_scatter(ref, [jnp.full((16,), i, jnp.int32)],
                               jnp.full((16,), v), mask=lane0)

        def row_body(r, _):
            cxr = splat_at(cx_v, r)
            cyr = splat_at(cy_v, r)
            czr = splat_at(cz_v, r)

            def seg_body(s, _):
                m = jnp.full((16,), inf)
                for j in range(_CPS):
                    base = s * _SEG + j * 16
                    xv = x_v[pl.ds(base, 16)]
                    yv = y_v[pl.ds(base, 16)]
                    zv = z_v[pl.ds(base, 16)]
                    dx = cxr - xv
                    dy = cyr - yv
                    dz = czr - zv
                    d2 = dx * dx + dy * dy + dz * dz
                    d2_v[pl.ds(base, 16)] = d2
                    m = jnp.minimum(m, d2)
                store_at(sm_v, s, jnp.min(m))
                return 0

            lax.fori_loop(0, nseg, seg_body, 0, unroll=False)

            def ext_body(e, _):
                bv = jnp.full((16,), inf)
                bi = jnp.full((16,), big)
                for j in range(nseg // 16):
                    v = sm_v[pl.ds(j * 16, 16)]
                    ids = lane + j * 16
                    take = v < bv
                    bv = jnp.where(take, v, bv)
                    bi = jnp.where(take, ids, bi)
                mval = jnp.min(bv)
                s = jnp.min(jnp.where(bv == mval, bi, big))

                def scan_body(j, carry):
                    cbv, cbi = carry
                    base = s * _SEG + j * 16
                    v = d2_v[pl.ds(base, 16)]
                    ids = lane + base
                    take = v < cbv
                    return (jnp.where(take, v, cbv), jnp.where(take, ids, cbi))

                cbv, cbi = lax.fori_loop(
                    0, _CPS, scan_body,
                    (jnp.full((16,), inf), jnp.full((16,), big)))
                cmval = jnp.min(cbv)
                idx = jnp.min(jnp.where(cbv == cmval, cbi, big))

                plsc.store_scatter(sel_v, [e * 4 + lane],
                                   jnp.full((16,), idx), mask=lane < 4)

                store_at(d2_v, idx, inf)

                def remin_body(j, m):
                    return jnp.minimum(m, d2_v[pl.ds(s * _SEG + j * 16, 16)])

                m = lax.fori_loop(0, _CPS, remin_body, jnp.full((16,), inf))
                store_at(sm_v, s, jnp.min(m))
                return 0

            lax.fori_loop(0, k, ext_body, 0, unroll=False)

            cc = jnp.where(feat == 0, cxr,
                           jnp.where(feat == 1, cyr,
                                     jnp.where(feat == 2, czr, 0.0)))

            def out_body(j, _):
                ids = sel_v[pl.ds(j * 16, 16)]
                vx = plsc.load_gather(x_v, [ids])
                vy = plsc.load_gather(y_v, [ids])
                vz = plsc.load_gather(z_v, [ids])
                vw = plsc.load_gather(w_v, [ids])
                val = jnp.where(feat == 0, vx,
                                jnp.where(feat == 1, vy,
                                          jnp.where(feat == 2, vz, vw)))
                out_v[pl.ds(r * (k * 4) + j * 16, 16)] = val - cc
                return 0

            lax.fori_loop(0, k * 4 // 16, out_body, 0, unroll=False)
            return 0

        lax.fori_loop(0, rows_per_w, row_body, 0, unroll=False)
        pltpu.sync_copy(out_v, out_hbm.at[pl.ds(wid * per_w, per_w)])

    return sc_knn(x, y, z, w, cx, cy, cz)


def kernel(points):
    B, N, C = points.shape
    x = points[:, :, 0]
    y = points[:, :, 1]
    z = points[:, :, 2]
    w = points[:, :, 3]

    cx, cy, cz = _fps(x, y, z, _G)
    centers = jnp.stack([cx, cy, cz], axis=-1)

    flat = _knn_gather_sc(x, y, z, w, cx, cy, cz, _G, _K)
    groups = flat.reshape(B, _G, _K, 4)
    return groups, centers


# speedup vs baseline: 25.3343x; 1.6743x over previous
import functools

import jax
import jax.numpy as jnp
from jax import lax
from jax.experimental import pallas as pl
from jax.experimental.pallas import tpu as pltpu
from jax.experimental.pallas import tpu_sc as plsc

_G = 512
_K = 32


def _fps_body(x_ref, y_ref, z_ref, md_in_ref, cx_ref, cy_ref, cz_ref,
              md_ref):
    B, N = x_ref.shape
    G = cx_ref.shape[1]
    x = x_ref[...]
    y = y_ref[...]
    z = z_ref[...]
    glanes = lax.broadcasted_iota(jnp.int32, (B, G), 1)
    md_ref[...] = md_in_ref[...]

    def body(i, _):
        md = md_ref[...]
        tx, ty, tz = x, y, z
        width = N
        while width > 1:
            width //= 2
            lo = md[:, :width]
            hi = md[:, width:]
            take = hi > lo
            md = jnp.where(take, hi, lo)
            tx = jnp.where(take, tx[:, width:], tx[:, :width])
            ty = jnp.where(take, ty[:, width:], ty[:, :width])
            tz = jnp.where(take, tz[:, width:], tz[:, :width])
        cx, cy, cz = tx, ty, tz
        gm = glanes == i
        cx_ref[...] = jnp.where(gm, cx, cx_ref[...])
        cy_ref[...] = jnp.where(gm, cy, cy_ref[...])
        cz_ref[...] = jnp.where(gm, cz, cz_ref[...])
        dx = x - cx
        dy = y - cy
        dz = z - cz
        d = dx * dx + dy * dy + dz * dz
        md_ref[...] = jnp.minimum(md_ref[...], d)
        return 0

    lax.fori_loop(0, G, body, 0)


def _fps(x, y, z, md, g):
    B, N = x.shape
    out = jax.ShapeDtypeStruct((B, g), jnp.float32)
    mdo = jax.ShapeDtypeStruct((B, N), jnp.float32)
    return pl.pallas_call(
        _fps_body,
        out_shape=[out, out, out, mdo],
    )(x, y, z, md)


_SEG = 256
_CPS = _SEG // 16


def _knn_gather_sc(x, y, z, w, cx, cy, cz, g, k):
    B, N = x.shape
    nw = 32
    rows_per_w = (B * g) // nw
    w_per_b = nw // B
    nseg = N // _SEG
    per_w = rows_per_w * k * 4
    mesh = plsc.VectorSubcoreMesh(core_axis_name="c", subcore_axis_name="s")
    big = jnp.int32(2 ** 30)
    inf = jnp.float32(jnp.inf)

    @functools.partial(
        pl.kernel,
        out_type=jax.ShapeDtypeStruct((nw * per_w,), jnp.float32),
        mesh=mesh,
        compiler_params=pltpu.CompilerParams(needs_layout_passes=False),
        scratch_types=[
            pltpu.VMEM((N,), jnp.float32),
            pltpu.VMEM((N,), jnp.float32),
            pltpu.VMEM((N,), jnp.float32),
            pltpu.VMEM((N,), jnp.float32),
            pltpu.VMEM((rows_per_w,), jnp.float32),
            pltpu.VMEM((rows_per_w,), jnp.float32),
            pltpu.VMEM((rows_per_w,), jnp.float32),
            pltpu.VMEM((N,), jnp.float32),
            pltpu.VMEM((N,), jnp.float32),
            pltpu.VMEM((nseg,), jnp.float32),
            pltpu.VMEM((nseg,), jnp.float32),
            pltpu.VMEM((k * 4,), jnp.int32),
            pltpu.VMEM((per_w,), jnp.float32),
        ],
    )
    def sc_knn(x_hbm, y_hbm, z_hbm, w_hbm, cx_hbm, cy_hbm, cz_hbm, out_hbm,
               x_v, y_v, z_v, w_v, cx_v, cy_v, cz_v, d2a_v, d2b_v, sma_v,
               smb_v, sel_v, out_v):
        wid = lax.axis_index("s") * 2 + lax.axis_index("c")
        b = wid // w_per_b
        g0 = (wid % w_per_b) * rows_per_w
        pltpu.sync_copy(x_hbm.at[b], x_v)
        pltpu.sync_copy(y_hbm.at[b], y_v)
        pltpu.sync_copy(z_hbm.at[b], z_v)
        pltpu.sync_copy(w_hbm.at[b], w_v)
        pltpu.sync_copy(cx_hbm.at[b, pl.ds(g0, rows_per_w)], cx_v)
        pltpu.sync_copy(cy_hbm.at[b, pl.ds(g0, rows_per_w)], cy_v)
        pltpu.sync_copy(cz_hbm.at[b, pl.ds(g0, rows_per_w)], cz_v)

        lane = jax.lax.iota(jnp.int32, 16)
        feat = lane % 4
        lane0 = lane == 0

        def splat_at(ref, i):
            return plsc.load_gather(ref, [jnp.full((16,), i, jnp.int32)])

        def store_at(ref, i, v):
            plsc.store_scatter(ref, [jnp.full((16,), i, jnp.int32)],
                               jnp.full((16,), v), mask=lane0)

        def select_and_emit(r, d2_v, sm_v, cxr, cyr, czr):
            def ext_body(e, _):
                bv = jnp.full((16,), inf)
                bi = jnp.full((16,), big)
                for j in range(nseg // 16):
                    v = sm_v[pl.ds(j * 16, 16)]
                    ids = lane + j * 16
                    take = v < bv
                    bv = jnp.where(take, v, bv)
                    bi = jnp.where(take, ids, bi)
                mval = jnp.min(bv)
                s = jnp.min(jnp.where(bv == mval, bi, big))

                cbv = jnp.full((16,), inf)
                cbi = jnp.full((16,), big)
                for j in range(_CPS):
                    base = s * _SEG + j * 16
                    v = d2_v[pl.ds(base, 16)]
                    ids = lane + base
                    take = v < cbv
                    cbv = jnp.where(take, v, cbv)
                    cbi = jnp.where(take, ids, cbi)
                cmval = jnp.min(cbv)
                idx = jnp.min(jnp.where(cbv == cmval, cbi, big))

                plsc.store_scatter(sel_v, [e * 4 + lane],
                                   jnp.full((16,), idx), mask=lane < 4)

                store_at(d2_v, idx, inf)
                m = jnp.full((16,), inf)
                for j in range(_CPS):
                    m = jnp.minimum(m, d2_v[pl.ds(s * _SEG + j * 16, 16)])
                store_at(sm_v, s, jnp.min(m))
                return 0

            lax.fori_loop(0, k, ext_body, 0, unroll=False)

            cc = jnp.where(feat == 0, cxr,
                           jnp.where(feat == 1, cyr,
                                     jnp.where(feat == 2, czr, 0.0)))

            def out_body(j, _):
                ids = sel_v[pl.ds(j * 16, 16)]
                vx = plsc.load_gather(x_v, [ids])
                vy = plsc.load_gather(y_v, [ids])
                vz = plsc.load_gather(z_v, [ids])
                vw = plsc.load_gather(w_v, [ids])
                val = jnp.where(feat == 0, vx,
                                jnp.where(feat == 1, vy,
                                          jnp.where(feat == 2, vz, vw)))
                out_v[pl.ds(r * (k * 4) + j * 16, 16)] = val - cc
                return 0

            lax.fori_loop(0, k * 4 // 16, out_body, 0, unroll=True)

        def row_body(r2, _):
            r0 = r2 * 2
            r1 = r0 + 1
            cx0 = splat_at(cx_v, r0)
            cy0 = splat_at(cy_v, r0)
            cz0 = splat_at(cz_v, r0)
            cx1 = splat_at(cx_v, r1)
            cy1 = splat_at(cy_v, r1)
            cz1 = splat_at(cz_v, r1)

            def seg_body(s, _):
                m0 = jnp.full((16,), inf)
                m1 = jnp.full((16,), inf)
                for j in range(_CPS):
                    base = s * _SEG + j * 16
                    xv = x_v[pl.ds(base, 16)]
                    yv = y_v[pl.ds(base, 16)]
                    zv = z_v[pl.ds(base, 16)]
                    dx0 = cx0 - xv
                    dy0 = cy0 - yv
                    dz0 = cz0 - zv
                    d20 = dx0 * dx0 + dy0 * dy0 + dz0 * dz0
                    dx1 = cx1 - xv
                    dy1 = cy1 - yv
                    dz1 = cz1 - zv
                    d21 = dx1 * dx1 + dy1 * dy1 + dz1 * dz1
                    d2a_v[pl.ds(base, 16)] = d20
                    d2b_v[pl.ds(base, 16)] = d21
                    m0 = jnp.minimum(m0, d20)
                    m1 = jnp.minimum(m1, d21)
                store_at(sma_v, s, jnp.min(m0))
                store_at(smb_v, s, jnp.min(m1))
                return 0

            lax.fori_loop(0, nseg, seg_body, 0, unroll=False)

            select_and_emit(r0, d2a_v, sma_v, cx0, cy0, cz0)
            select_and_emit(r1, d2b_v, smb_v, cx1, cy1, cz1)
            return 0

        lax.fori_loop(0, rows_per_w // 2, row_body, 0, unroll=False)
        pltpu.sync_copy(out_v, out_hbm.at[pl.ds(wid * per_w, per_w)])

    return sc_knn(x, y, z, w, cx, cy, cz)


_STAGES = 2


def kernel(points):
    B, N, C = points.shape
    x = points[:, :, 0]
    y = points[:, :, 1]
    z = points[:, :, 2]
    w = points[:, :, 3]

    gs = _G // _STAGES
    md = jnp.full((B, N), 1e10, jnp.float32)
    group_parts = []
    center_parts = []
    for _ in range(_STAGES):
        cx, cy, cz, md = _fps(x, y, z, md, gs)
        center_parts.append(jnp.stack([cx, cy, cz], axis=-1))
        flat = _knn_gather_sc(x, y, z, w, cx, cy, cz, gs, _K)
        group_parts.append(flat.reshape(B, gs, _K, 4))
    groups = jnp.concatenate(group_parts, axis=1)
    centers = jnp.concatenate(center_parts, axis=1)
    return groups, centers


# speedup vs baseline: 27.9301x; 1.1025x over previous
import functools

import jax
import jax.numpy as jnp
from jax import lax
from jax.experimental import pallas as pl
from jax.experimental.pallas import tpu as pltpu
from jax.experimental.pallas import tpu_sc as plsc

_G = 512
_K = 32


def _fps_body(x_ref, y_ref, z_ref, md_in_ref, cx_ref, cy_ref, cz_ref,
              md_ref):
    B, N = x_ref.shape
    G = cx_ref.shape[1]
    x = x_ref[...]
    y = y_ref[...]
    z = z_ref[...]
    glanes = lax.broadcasted_iota(jnp.int32, (B, G), 1)
    md_ref[...] = md_in_ref[...]

    def body(i, _):
        md = md_ref[...]
        tx, ty, tz = x, y, z
        width = N
        while width > 1:
            width //= 2
            lo = md[:, :width]
            hi = md[:, width:]
            take = hi > lo
            md = jnp.where(take, hi, lo)
            tx = jnp.where(take, tx[:, width:], tx[:, :width])
            ty = jnp.where(take, ty[:, width:], ty[:, :width])
            tz = jnp.where(take, tz[:, width:], tz[:, :width])
        cx, cy, cz = tx, ty, tz
        gm = glanes == i
        cx_ref[...] = jnp.where(gm, cx, cx_ref[...])
        cy_ref[...] = jnp.where(gm, cy, cy_ref[...])
        cz_ref[...] = jnp.where(gm, cz, cz_ref[...])
        dx = x - cx
        dy = y - cy
        dz = z - cz
        d = dx * dx + dy * dy + dz * dz
        md_ref[...] = jnp.minimum(md_ref[...], d)
        return 0

    lax.fori_loop(0, G, body, 0)


def _fps(x, y, z, md, g):
    B, N = x.shape
    out = jax.ShapeDtypeStruct((B, g), jnp.float32)
    mdo = jax.ShapeDtypeStruct((B, N), jnp.float32)
    return pl.pallas_call(
        _fps_body,
        out_shape=[out, out, out, mdo],
    )(x, y, z, md)


_SEG = 256
_CPS = _SEG // 16


def _knn_gather_sc(x, y, z, w, cx, cy, cz, g, k):
    B, N = x.shape
    nw = 32
    rows_per_w = (B * g) // nw
    w_per_b = nw // B
    nseg = N // _SEG
    per_w = rows_per_w * k * 4
    mesh = plsc.VectorSubcoreMesh(core_axis_name="c", subcore_axis_name="s")
    big = jnp.int32(2 ** 30)
    inf = jnp.float32(jnp.inf)

    @functools.partial(
        pl.kernel,
        out_type=jax.ShapeDtypeStruct((nw * per_w,), jnp.float32),
        mesh=mesh,
        compiler_params=pltpu.CompilerParams(needs_layout_passes=False),
        scratch_types=[
            pltpu.VMEM((N,), jnp.float32),
            pltpu.VMEM((N,), jnp.float32),
            pltpu.VMEM((N,), jnp.float32),
            pltpu.VMEM((N,), jnp.float32),
            pltpu.VMEM((rows_per_w,), jnp.float32),
            pltpu.VMEM((rows_per_w,), jnp.float32),
            pltpu.VMEM((rows_per_w,), jnp.float32),
            pltpu.VMEM((N,), jnp.float32),
            pltpu.VMEM((N,), jnp.float32),
            pltpu.VMEM((nseg,), jnp.float32),
            pltpu.VMEM((nseg,), jnp.float32),
            pltpu.VMEM((k * 4,), jnp.int32),
            pltpu.VMEM((per_w,), jnp.float32),
        ],
    )
    def sc_knn(x_hbm, y_hbm, z_hbm, w_hbm, cx_hbm, cy_hbm, cz_hbm, out_hbm,
               x_v, y_v, z_v, w_v, cx_v, cy_v, cz_v, d2a_v, d2b_v, sma_v,
               smb_v, sel_v, out_v):
        wid = lax.axis_index("s") * 2 + lax.axis_index("c")
        b = wid // w_per_b
        g0 = (wid % w_per_b) * rows_per_w
        pltpu.sync_copy(x_hbm.at[b], x_v)
        pltpu.sync_copy(y_hbm.at[b], y_v)
        pltpu.sync_copy(z_hbm.at[b], z_v)
        pltpu.sync_copy(w_hbm.at[b], w_v)
        pltpu.sync_copy(cx_hbm.at[b, pl.ds(g0, rows_per_w)], cx_v)
        pltpu.sync_copy(cy_hbm.at[b, pl.ds(g0, rows_per_w)], cy_v)
        pltpu.sync_copy(cz_hbm.at[b, pl.ds(g0, rows_per_w)], cz_v)

        lane = jax.lax.iota(jnp.int32, 16)
        feat = lane % 4
        lane0 = lane == 0

        def splat_at(ref, i):
            return plsc.load_gather(ref, [jnp.full((16,), i, jnp.int32)])

        def store_at(ref, i, v):
            plsc.store_scatter(ref, [jnp.full((16,), i, jnp.int32)],
                               jnp.full((16,), v), mask=lane0)

        def select_and_emit(r, d2_v, sm_v, cxr, cyr, czr):
            def ext_body(e, _):
                bv = jnp.full((16,), inf)
                bi = jnp.full((16,), big)
                for j in range(nseg // 16):
                    v = sm_v[pl.ds(j * 16, 16)]
                    ids = lane + j * 16
                    take = v < bv
                    bv = jnp.where(take, v, bv)
                    bi = jnp.where(take, ids, bi)
                mval = jnp.min(bv)
                s = jnp.min(jnp.where(bv == mval, bi, big))

                cbv = jnp.full((16,), inf)
                cbi = jnp.full((16,), big)
                for j in range(_CPS):
                    base = s * _SEG + j * 16
                    v = d2_v[pl.ds(base, 16)]
                    ids = lane + base
                    take = v < cbv
                    cbv = jnp.where(take, v, cbv)
                    cbi = jnp.where(take, ids, cbi)
                cmval = jnp.min(cbv)
                idx = jnp.min(jnp.where(cbv == cmval, cbi, big))

                plsc.store_scatter(sel_v, [e * 4 + lane],
                                   jnp.full((16,), idx), mask=lane < 4)

                store_at(d2_v, idx, inf)
                m = jnp.full((16,), inf)
                for j in range(_CPS):
                    m = jnp.minimum(m, d2_v[pl.ds(s * _SEG + j * 16, 16)])
                store_at(sm_v, s, jnp.min(m))
                return 0

            lax.fori_loop(0, k, ext_body, 0, unroll=False)

            cc = jnp.where(feat == 0, cxr,
                           jnp.where(feat == 1, cyr,
                                     jnp.where(feat == 2, czr, 0.0)))

            def out_body(j, _):
                ids = sel_v[pl.ds(j * 16, 16)]
                vx = plsc.load_gather(x_v, [ids])
                vy = plsc.load_gather(y_v, [ids])
                vz = plsc.load_gather(z_v, [ids])
                vw = plsc.load_gather(w_v, [ids])
                val = jnp.where(feat == 0, vx,
                                jnp.where(feat == 1, vy,
                                          jnp.where(feat == 2, vz, vw)))
                out_v[pl.ds(r * (k * 4) + j * 16, 16)] = val - cc
                return 0

            lax.fori_loop(0, k * 4 // 16, out_body, 0, unroll=True)

        def row_body(r2, _):
            r0 = r2 * 2
            r1 = r0 + 1
            cx0 = splat_at(cx_v, r0)
            cy0 = splat_at(cy_v, r0)
            cz0 = splat_at(cz_v, r0)
            cx1 = splat_at(cx_v, r1)
            cy1 = splat_at(cy_v, r1)
            cz1 = splat_at(cz_v, r1)

            def seg_body(s, _):
                m0 = jnp.full((16,), inf)
                m1 = jnp.full((16,), inf)
                for j in range(_CPS):
                    base = s * _SEG + j * 16
                    xv = x_v[pl.ds(base, 16)]
                    yv = y_v[pl.ds(base, 16)]
                    zv = z_v[pl.ds(base, 16)]
                    dx0 = cx0 - xv
                    dy0 = cy0 - yv
                    dz0 = cz0 - zv
                    d20 = dx0 * dx0 + dy0 * dy0 + dz0 * dz0
                    dx1 = cx1 - xv
                    dy1 = cy1 - yv
                    dz1 = cz1 - zv
                    d21 = dx1 * dx1 + dy1 * dy1 + dz1 * dz1
                    d2a_v[pl.ds(base, 16)] = d20
                    d2b_v[pl.ds(base, 16)] = d21
                    m0 = jnp.minimum(m0, d20)
                    m1 = jnp.minimum(m1, d21)
                store_at(sma_v, s, jnp.min(m0))
                store_at(smb_v, s, jnp.min(m1))
                return 0

            lax.fori_loop(0, nseg, seg_body, 0, unroll=False)

            select_and_emit(r0, d2a_v, sma_v, cx0, cy0, cz0)
            select_and_emit(r1, d2b_v, smb_v, cx1, cy1, cz1)
            return 0

        lax.fori_loop(0, rows_per_w // 2, row_body, 0, unroll=False)
        pltpu.sync_copy(out_v, out_hbm.at[pl.ds(wid * per_w, per_w)])

    return sc_knn(x, y, z, w, cx, cy, cz)


_STAGES = 4


def kernel(points):
    B, N, C = points.shape
    x = points[:, :, 0]
    y = points[:, :, 1]
    z = points[:, :, 2]
    w = points[:, :, 3]

    gs = _G // _STAGES
    md = jnp.full((B, N), 1e10, jnp.float32)
    group_parts = []
    center_parts = []
    for _ in range(_STAGES):
        cx, cy, cz, md = _fps(x, y, z, md, gs)
        center_parts.append(jnp.stack([cx, cy, cz], axis=-1))
        flat = _knn_gather_sc(x, y, z, w, cx, cy, cz, gs, _K)
        group_parts.append(flat.reshape(B, gs, _K, 4))
    groups = jnp.concatenate(group_parts, axis=1)
    centers = jnp.concatenate(center_parts, axis=1)
    return groups, centers


# speedup vs baseline: 28.1025x; 1.0062x over previous
import functools

import jax
import jax.numpy as jnp
from jax import lax
from jax.experimental import pallas as pl
from jax.experimental.pallas import tpu as pltpu
from jax.experimental.pallas import tpu_sc as plsc

_G = 512
_K = 32


def _fps_body(x_ref, y_ref, z_ref, md_in_ref, cx_ref, cy_ref, cz_ref,
              md_ref):
    B, N = x_ref.shape
    G = cx_ref.shape[1]
    x = x_ref[...]
    y = y_ref[...]
    z = z_ref[...]
    glanes = lax.broadcasted_iota(jnp.int32, (B, G), 1)
    md_ref[...] = md_in_ref[...]

    def body(i, _):
        md = md_ref[...]
        tx, ty, tz = x, y, z
        width = N
        while width > 1:
            width //= 2
            lo = md[:, :width]
            hi = md[:, width:]
            take = hi > lo
            md = jnp.where(take, hi, lo)
            tx = jnp.where(take, tx[:, width:], tx[:, :width])
            ty = jnp.where(take, ty[:, width:], ty[:, :width])
            tz = jnp.where(take, tz[:, width:], tz[:, :width])
        cx, cy, cz = tx, ty, tz
        gm = glanes == i
        cx_ref[...] = jnp.where(gm, cx, cx_ref[...])
        cy_ref[...] = jnp.where(gm, cy, cy_ref[...])
        cz_ref[...] = jnp.where(gm, cz, cz_ref[...])
        dx = x - cx
        dy = y - cy
        dz = z - cz
        d = dx * dx + dy * dy + dz * dz
        md_ref[...] = jnp.minimum(md_ref[...], d)
        return 0

    lax.fori_loop(0, G, body, 0)


def _fps(x, y, z, md, g):
    B, N = x.shape
    out = jax.ShapeDtypeStruct((B, g), jnp.float32)
    mdo = jax.ShapeDtypeStruct((B, N), jnp.float32)
    return pl.pallas_call(
        _fps_body,
        out_shape=[out, out, out, mdo],
    )(x, y, z, md)


_SEG = 256
_CPS = _SEG // 16


def _knn_gather_sc(x, y, z, w, cx, cy, cz, g, k):
    B, N = x.shape
    nw = 32
    rows_per_w = (B * g) // nw
    w_per_b = nw // B
    nseg = N // _SEG
    per_w = rows_per_w * k * 4
    mesh = plsc.VectorSubcoreMesh(core_axis_name="c", subcore_axis_name="s")
    big = jnp.int32(2 ** 30)
    inf = jnp.float32(jnp.inf)

    @functools.partial(
        pl.kernel,
        out_type=jax.ShapeDtypeStruct((nw * per_w,), jnp.float32),
        mesh=mesh,
        compiler_params=pltpu.CompilerParams(needs_layout_passes=False),
        scratch_types=[
            pltpu.VMEM((N,), jnp.float32),
            pltpu.VMEM((N,), jnp.float32),
            pltpu.VMEM((N,), jnp.float32),
            pltpu.VMEM((N,), jnp.float32),
            pltpu.VMEM((rows_per_w,), jnp.float32),
            pltpu.VMEM((rows_per_w,), jnp.float32),
            pltpu.VMEM((rows_per_w,), jnp.float32),
            pltpu.VMEM((N,), jnp.float32),
            pltpu.VMEM((N,), jnp.float32),
            pltpu.VMEM((nseg,), jnp.float32),
            pltpu.VMEM((nseg,), jnp.float32),
            pltpu.VMEM((k * 4,), jnp.int32),
            pltpu.VMEM((per_w,), jnp.float32),
        ],
    )
    def sc_knn(x_hbm, y_hbm, z_hbm, w_hbm, cx_hbm, cy_hbm, cz_hbm, out_hbm,
               x_v, y_v, z_v, w_v, cx_v, cy_v, cz_v, d2a_v, d2b_v, sma_v,
               smb_v, sel_v, out_v):
        wid = lax.axis_index("s") * 2 + lax.axis_index("c")
        b = wid // w_per_b
        g0 = (wid % w_per_b) * rows_per_w
        pltpu.sync_copy(x_hbm.at[b], x_v)
        pltpu.sync_copy(y_hbm.at[b], y_v)
        pltpu.sync_copy(z_hbm.at[b], z_v)
        pltpu.sync_copy(w_hbm.at[b], w_v)
        pltpu.sync_copy(cx_hbm.at[b, pl.ds(g0, rows_per_w)], cx_v)
        pltpu.sync_copy(cy_hbm.at[b, pl.ds(g0, rows_per_w)], cy_v)
        pltpu.sync_copy(cz_hbm.at[b, pl.ds(g0, rows_per_w)], cz_v)

        lane = jax.lax.iota(jnp.int32, 16)
        feat = lane % 4
        lane0 = lane == 0

        def splat_at(ref, i):
            return plsc.load_gather(ref, [jnp.full((16,), i, jnp.int32)])

        def store_at(ref, i, v):
            plsc.store_scatter(ref, [jnp.full((16,), i, jnp.int32)],
                               jnp.full((16,), v), mask=lane0)

        def select_and_emit(r, d2_v, sm_v, cxr, cyr, czr):
            def ext_body(e, _):
                bv = jnp.full((16,), inf)
                bi = jnp.full((16,), big)
                for j in range(nseg // 16):
                    v = sm_v[pl.ds(j * 16, 16)]
                    ids = lane + j * 16
                    take = v < bv
                    bv = jnp.where(take, v, bv)
                    bi = jnp.where(take, ids, bi)
                mval = jnp.min(bv)
                s = jnp.min(jnp.where(bv == mval, bi, big))

                cbv = jnp.full((16,), inf)
                cbi = jnp.full((16,), big)
                for j in range(_CPS):
                    base = s * _SEG + j * 16
                    v = d2_v[pl.ds(base, 16)]
                    ids = lane + base
                    take = v < cbv
                    cbv = jnp.where(take, v, cbv)
                    cbi = jnp.where(take, ids, cbi)
                cmval = jnp.min(cbv)
                idx = jnp.min(jnp.where(cbv == cmval, cbi, big))

                plsc.store_scatter(sel_v, [e * 4 + lane],
                                   jnp.full((16,), idx), mask=lane < 4)

                store_at(d2_v, idx, inf)
                m = jnp.full((16,), inf)
                for j in range(_CPS):
                    m = jnp.minimum(m, d2_v[pl.ds(s * _SEG + j * 16, 16)])
                store_at(sm_v, s, jnp.min(m))
                return 0

            lax.fori_loop(0, k, ext_body, 0, unroll=False)

            cc = jnp.where(feat == 0, cxr,
                           jnp.where(feat == 1, cyr,
                                     jnp.where(feat == 2, czr, 0.0)))

            def out_body(j, _):
                ids = sel_v[pl.ds(j * 16, 16)]
                vx = plsc.load_gather(x_v, [ids])
                vy = plsc.load_gather(y_v, [ids])
                vz = plsc.load_gather(z_v, [ids])
                vw = plsc.load_gather(w_v, [ids])
                val = jnp.where(feat == 0, vx,
                                jnp.where(feat == 1, vy,
                                          jnp.where(feat == 2, vz, vw)))
                out_v[pl.ds(r * (k * 4) + j * 16, 16)] = val - cc
                return 0

            lax.fori_loop(0, k * 4 // 16, out_body, 0, unroll=True)

        def row_body(r2, _):
            r0 = r2 * 2
            r1 = r0 + 1
            cx0 = splat_at(cx_v, r0)
            cy0 = splat_at(cy_v, r0)
            cz0 = splat_at(cz_v, r0)
            cx1 = splat_at(cx_v, r1)
            cy1 = splat_at(cy_v, r1)
            cz1 = splat_at(cz_v, r1)

            def seg_body(s, _):
                m0 = jnp.full((16,), inf)
                m1 = jnp.full((16,), inf)
                for j in range(_CPS):
                    base = s * _SEG + j * 16
                    xv = x_v[pl.ds(base, 16)]
                    yv = y_v[pl.ds(base, 16)]
                    zv = z_v[pl.ds(base, 16)]
                    dx0 = cx0 - xv
                    dy0 = cy0 - yv
                    dz0 = cz0 - zv
                    d20 = dx0 * dx0 + dy0 * dy0 + dz0 * dz0
                    dx1 = cx1 - xv
                    dy1 = cy1 - yv
                    dz1 = cz1 - zv
                    d21 = dx1 * dx1 + dy1 * dy1 + dz1 * dz1
                    d2a_v[pl.ds(base, 16)] = d20
                    d2b_v[pl.ds(base, 16)] = d21
                    m0 = jnp.minimum(m0, d20)
                    m1 = jnp.minimum(m1, d21)
                store_at(sma_v, s, jnp.min(m0))
                store_at(smb_v, s, jnp.min(m1))
                return 0

            lax.fori_loop(0, nseg, seg_body, 0, unroll=False)

            select_and_emit(r0, d2a_v, sma_v, cx0, cy0, cz0)
            select_and_emit(r1, d2b_v, smb_v, cx1, cy1, cz1)
            return 0

        lax.fori_loop(0, rows_per_w // 2, row_body, 0, unroll=False)
        pltpu.sync_copy(out_v, out_hbm.at[pl.ds(wid * per_w, per_w)])

    return sc_knn(x, y, z, w, cx, cy, cz)


_STAGES = 8


def kernel(points):
    B, N, C = points.shape
    x = points[:, :, 0]
    y = points[:, :, 1]
    z = points[:, :, 2]
    w = points[:, :, 3]

    gs = _G // _STAGES
    md = jnp.full((B, N), 1e10, jnp.float32)
    group_parts = []
    center_parts = []
    for _ in range(_STAGES):
        cx, cy, cz, md = _fps(x, y, z, md, gs)
        center_parts.append(jnp.stack([cx, cy, cz], axis=-1))
        flat = _knn_gather_sc(x, y, z, w, cx, cy, cz, gs, _K)
        group_parts.append(flat.reshape(B, gs, _K, 4))
    groups = jnp.concatenate(group_parts, axis=1)
    centers = jnp.concatenate(center_parts, axis=1)
    return groups, centers


# speedup vs baseline: 40.1885x; 1.4301x over previous
import functools

import jax
import jax.numpy as jnp
from jax import lax
from jax.experimental import pallas as pl
from jax.experimental.pallas import tpu as pltpu
from jax.experimental.pallas import tpu_sc as plsc

_G = 512
_K = 32


def _fps_body(x_ref, y_ref, z_ref, md_in_ref, cx_ref, cy_ref, cz_ref,
              md_ref):
    B, N = x_ref.shape
    G = cx_ref.shape[1]
    x = x_ref[...]
    y = y_ref[...]
    z = z_ref[...]
    glanes = lax.broadcasted_iota(jnp.int32, (B, G), 1)
    md_ref[...] = md_in_ref[...]

    def body(i, _):
        md = md_ref[...]
        tx, ty, tz = x, y, z
        width = N
        while width > 1:
            width //= 2
            lo = md[:, :width]
            hi = md[:, width:]
            take = hi > lo
            md = jnp.where(take, hi, lo)
            tx = jnp.where(take, tx[:, width:], tx[:, :width])
            ty = jnp.where(take, ty[:, width:], ty[:, :width])
            tz = jnp.where(take, tz[:, width:], tz[:, :width])
        cx, cy, cz = tx, ty, tz
        gm = glanes == i
        cx_ref[...] = jnp.where(gm, cx, cx_ref[...])
        cy_ref[...] = jnp.where(gm, cy, cy_ref[...])
        cz_ref[...] = jnp.where(gm, cz, cz_ref[...])
        dx = x - cx
        dy = y - cy
        dz = z - cz
        d = dx * dx + dy * dy + dz * dz
        md_ref[...] = jnp.minimum(md_ref[...], d)
        return 0

    lax.fori_loop(0, G, body, 0)


def _fps(x, y, z, md, g):
    B, N = x.shape
    out = jax.ShapeDtypeStruct((B, g), jnp.float32)
    mdo = jax.ShapeDtypeStruct((B, N), jnp.float32)
    return pl.pallas_call(
        _fps_body,
        out_shape=[out, out, out, mdo],
    )(x, y, z, md)


_SEG = 64
_CPS = _SEG // 16
_BLK = 256
_SPB = _BLK // _SEG


def _knn_gather_sc(x, y, z, w, cx, cy, cz, g, k):
    B, N = x.shape
    nw = 32
    rows_per_w = (B * g) // nw
    w_per_b = nw // B
    nseg = N // _SEG
    per_w = rows_per_w * k * 4
    mesh = plsc.VectorSubcoreMesh(core_axis_name="c", subcore_axis_name="s")
    big = jnp.int32(2 ** 30)
    inf = jnp.float32(jnp.inf)

    @functools.partial(
        pl.kernel,
        out_type=jax.ShapeDtypeStruct((nw * per_w,), jnp.float32),
        mesh=mesh,
        compiler_params=pltpu.CompilerParams(needs_layout_passes=False),
        scratch_types=[
            pltpu.VMEM((N,), jnp.float32),
            pltpu.VMEM((N,), jnp.float32),
            pltpu.VMEM((N,), jnp.float32),
            pltpu.VMEM((N,), jnp.float32),
            pltpu.VMEM((rows_per_w,), jnp.float32),
            pltpu.VMEM((rows_per_w,), jnp.float32),
            pltpu.VMEM((rows_per_w,), jnp.float32),
            pltpu.VMEM((N,), jnp.float32),
            pltpu.VMEM((N,), jnp.float32),
            pltpu.VMEM((nseg,), jnp.float32),
            pltpu.VMEM((nseg,), jnp.float32),
            pltpu.VMEM((k * 4,), jnp.int32),
            pltpu.VMEM((per_w,), jnp.float32),
        ],
    )
    def sc_knn(x_hbm, y_hbm, z_hbm, w_hbm, cx_hbm, cy_hbm, cz_hbm, out_hbm,
               x_v, y_v, z_v, w_v, cx_v, cy_v, cz_v, d2a_v, d2b_v, sma_v,
               smb_v, sel_v, out_v):
        wid = lax.axis_index("s") * 2 + lax.axis_index("c")
        b = wid // w_per_b
        g0 = (wid % w_per_b) * rows_per_w
        pltpu.sync_copy(x_hbm.at[b], x_v)
        pltpu.sync_copy(y_hbm.at[b], y_v)
        pltpu.sync_copy(z_hbm.at[b], z_v)
        pltpu.sync_copy(w_hbm.at[b], w_v)
        pltpu.sync_copy(cx_hbm.at[b, pl.ds(g0, rows_per_w)], cx_v)
        pltpu.sync_copy(cy_hbm.at[b, pl.ds(g0, rows_per_w)], cy_v)
        pltpu.sync_copy(cz_hbm.at[b, pl.ds(g0, rows_per_w)], cz_v)

        lane = jax.lax.iota(jnp.int32, 16)
        feat = lane % 4
        lane0 = lane == 0

        def splat_at(ref, i):
            return plsc.load_gather(ref, [jnp.full((16,), i, jnp.int32)])

        def store_at(ref, i, v):
            plsc.store_scatter(ref, [jnp.full((16,), i, jnp.int32)],
                               jnp.full((16,), v), mask=lane0)

        def select_and_emit(r, d2_v, sm_v, cxr, cyr, czr):
            def ext_body(e, _):
                bv = jnp.full((16,), inf)
                bi = jnp.full((16,), big)
                for j in range(nseg // 16):
                    v = sm_v[pl.ds(j * 16, 16)]
                    ids = lane + j * 16
                    take = v < bv
                    bv = jnp.where(take, v, bv)
                    bi = jnp.where(take, ids, bi)
                mval = jnp.min(bv)
                s = jnp.min(jnp.where(bv == mval, bi, big))

                cbv = jnp.full((16,), inf)
                cbi = jnp.full((16,), big)
                for j in range(_CPS):
                    base = s * _SEG + j * 16
                    v = d2_v[pl.ds(base, 16)]
                    ids = lane + base
                    take = v < cbv
                    cbv = jnp.where(take, v, cbv)
                    cbi = jnp.where(take, ids, cbi)
                cmval = jnp.min(cbv)
                idx = jnp.min(jnp.where(cbv == cmval, cbi, big))

                plsc.store_scatter(sel_v, [e * 4 + lane],
                                   jnp.full((16,), idx), mask=lane < 4)

                store_at(d2_v, idx, inf)
                m = jnp.full((16,), inf)
                for j in range(_CPS):
                    m = jnp.minimum(m, d2_v[pl.ds(s * _SEG + j * 16, 16)])
                store_at(sm_v, s, jnp.min(m))
                return 0

            lax.fori_loop(0, k, ext_body, 0, unroll=False)

            cc = jnp.where(feat == 0, cxr,
                           jnp.where(feat == 1, cyr,
                                     jnp.where(feat == 2, czr, 0.0)))

            def out_body(j, _):
                ids = sel_v[pl.ds(j * 16, 16)]
                vx = plsc.load_gather(x_v, [ids])
                vy = plsc.load_gather(y_v, [ids])
                vz = plsc.load_gather(z_v, [ids])
                vw = plsc.load_gather(w_v, [ids])
                val = jnp.where(feat == 0, vx,
                                jnp.where(feat == 1, vy,
                                          jnp.where(feat == 2, vz, vw)))
                out_v[pl.ds(r * (k * 4) + j * 16, 16)] = val - cc
                return 0

            lax.fori_loop(0, k * 4 // 16, out_body, 0, unroll=True)

        def row_body(r2, _):
            r0 = r2 * 2
            cs = []
            for q in range(2):
                cs.append((splat_at(cx_v, r0 + q), splat_at(cy_v, r0 + q),
                           splat_at(cz_v, r0 + q)))
            d2s = (d2a_v, d2b_v)
            sms = (sma_v, smb_v)

            def seg_body(s, _):
                ms = [[jnp.full((16,), inf) for _ in range(_SPB)]
                      for _ in range(2)]
                for j in range(_BLK // 16):
                    base = s * _BLK + j * 16
                    xv = x_v[pl.ds(base, 16)]
                    yv = y_v[pl.ds(base, 16)]
                    zv = z_v[pl.ds(base, 16)]
                    for q in range(2):
                        cxq, cyq, czq = cs[q]
                        dx = cxq - xv
                        dy = cyq - yv
                        dz = czq - zv
                        d2 = dx * dx + dy * dy + dz * dz
                        d2s[q][pl.ds(base, 16)] = d2
                        ms[q][j // _CPS] = jnp.minimum(ms[q][j // _CPS], d2)
                for q in range(2):
                    for t in range(_SPB):
                        store_at(sms[q], s * _SPB + t, jnp.min(ms[q][t]))
                return 0

            lax.fori_loop(0, N // _BLK, seg_body, 0, unroll=False)

            for q in range(2):
                select_and_emit(r0 + q, d2s[q], sms[q], *cs[q])
            return 0

        lax.fori_loop(0, rows_per_w // 2, row_body, 0, unroll=False)
        pltpu.sync_copy(out_v, out_hbm.at[pl.ds(wid * per_w, per_w)])

    return sc_knn(x, y, z, w, cx, cy, cz)


_STAGES = 8


def kernel(points):
    B, N, C = points.shape
    x = points[:, :, 0]
    y = points[:, :, 1]
    z = points[:, :, 2]
    w = points[:, :, 3]

    gs = _G // _STAGES
    md = jnp.full((B, N), 1e10, jnp.float32)
    group_parts = []
    center_parts = []
    for _ in range(_STAGES):
        cx, cy, cz, md = _fps(x, y, z, md, gs)
        center_parts.append(jnp.stack([cx, cy, cz], axis=-1))
        flat = _knn_gather_sc(x, y, z, w, cx, cy, cz, gs, _K)
        group_parts.append(flat.reshape(B, gs, _K, 4))
    groups = jnp.concatenate(group_parts, axis=1)
    centers = jnp.concatenate(center_parts, axis=1)
    return groups, centers
